# baseline (speedup 1.0000x reference)
import functools

import jax
import jax.numpy as jnp
from jax import lax
from jax.experimental import pallas as pl
from jax.experimental.pallas import tpu as pltpu
from jax.experimental.pallas import tpu_sc as plsc

_N_VIDEO = 100000
_N_MU = 8
_OUT_DIM = 64
_BATCH = 16384
_P = 4
_PL = 128
_MOM = 0.9

_NC = 2
_NS = 16
_NW = _NC * _NS
_B_PER_W = _BATCH // _NW
_CHUNK = 64
_NCHUNK = _B_PER_W // _CHUNK
_LANE = 16

_V_BLOCK = 4096
_NIDV = _BATCH // _LANE


def _worker_id():
    return lax.axis_index("s") * _NC + lax.axis_index("c")


def _winner_body(vid_hbm, w_hbm, vidx_v, wtab_v):
    w = _worker_id()
    lo = w * _V_BLOCK
    pltpu.sync_copy(vid_hbm, vidx_v)

    neg1 = jnp.full((_LANE,), -1, jnp.int32)

    @pl.loop(0, _V_BLOCK // _LANE)
    def _init(i):
        wtab_v[pl.ds(i * _LANE, _LANE)] = neg1

    iota = lax.iota(jnp.int32, _LANE)

    @pl.loop(0, _NIDV)
    def _scan(i):
        v = vidx_v[pl.ds(i * _LANE, _LANE)]
        b = i * _LANE + iota
        r = v - lo
        m = (r >= 0) & (r < _V_BLOCK)
        r = jnp.where(m, r, 0)
        _, last = plsc.scan_count(v, m)
        plsc.store_scatter(wtab_v, [r], b, mask=m & last)

    pltpu.sync_copy(wtab_v, w_hbm.at[pl.ds(lo, _V_BLOCK)])


def _gather_blend_body(i_hbm, vid_hbm, wflat_hbm, q_ref, upd_hbm,
                       idx_v, bw_v, qbuf, ibuf, gsem, isem, wsem, ssem):
    w = _worker_id()
    base = w * _B_PER_W
    pltpu.sync_copy(vid_hbm.at[w], idx_v)

    for j in range(_NCHUNK):
        cq = pltpu.async_copy(q_ref.at[idx_v.at[j]], qbuf, gsem)
        pltpu.async_copy(wflat_hbm.at[idx_v.at[j]], bw_v, wsem).wait()
        ci = pltpu.async_copy(i_hbm.at[bw_v], ibuf, isem)
        cq.wait()
        ci.wait()

        @pl.loop(0, _CHUNK * _P)
        def _blend(i):
            r = i // _P
            pp = i % _P
            for c in range(_PL // _LANE):
                q = qbuf[r, pp, pl.ds(c * _LANE, _LANE)]
                x = ibuf[r, pp, pl.ds(c * _LANE, _LANE)]
                qbuf[r, pp, pl.ds(c * _LANE, _LANE)] = (
                    q * _MOM + x * (1.0 - _MOM))

        pltpu.async_copy(
            qbuf, upd_hbm.at[pl.ds(base + j * _CHUNK, _CHUNK)], ssem).wait()


def _scatter_body(upd_hbm, vid_hbm, q_ref, idx_v, buf, ssem):
    w = _worker_id()
    base = w * _B_PER_W
    pltpu.sync_copy(vid_hbm.at[w], idx_v)

    for j in range(_NCHUNK):
        pltpu.sync_copy(upd_hbm.at[pl.ds(base + j * _CHUNK, _CHUNK)], buf)
        pltpu.async_copy(buf, q_ref.at[idx_v.at[j]], ssem).wait()


@functools.cache
def _get_kernels():
    mesh = plsc.VectorSubcoreMesh(
        core_axis_name="c", subcore_axis_name="s", num_cores=_NC,
        num_subcores=_NS)
    winner = pl.kernel(
        _winner_body,
        out_type=jax.ShapeDtypeStruct((_NW * _V_BLOCK,), jnp.int32),
        mesh=mesh,
        compiler_params=pltpu.CompilerParams(needs_layout_passes=False),
        scratch_types=[
            pltpu.VMEM((_BATCH,), jnp.int32),
            pltpu.VMEM((_V_BLOCK,), jnp.int32),
        ],
    )
    gather_blend = pl.kernel(
        _gather_blend_body,
        out_type=jax.ShapeDtypeStruct((_BATCH, _P, _PL), jnp.float32),
        mesh=mesh,
        scratch_types=[
            pltpu.VMEM((_NCHUNK, _CHUNK), jnp.int32),
            pltpu.VMEM((_CHUNK,), jnp.int32),
            pltpu.VMEM((_CHUNK, _P, _PL), jnp.float32),
            pltpu.VMEM((_CHUNK, _P, _PL), jnp.float32),
            pltpu.SemaphoreType.DMA,
            pltpu.SemaphoreType.DMA,
            pltpu.SemaphoreType.DMA,
            pltpu.SemaphoreType.DMA,
        ],
    )
    scatter = pl.kernel(
        _scatter_body,
        out_type=(),
        mesh=mesh,
        scratch_types=[
            pltpu.VMEM((_NCHUNK, _CHUNK), jnp.int32),
            pltpu.VMEM((_CHUNK, _P, _PL), jnp.float32),
            pltpu.SemaphoreType.DMA,
        ],
    )
    return winner, gather_blend, scatter


@jax.jit
def kernel(queue, inp, vid_idx):
    winner, gather_blend, scatter = _get_kernels()
    iflat = inp.reshape(_BATCH, _P, _PL)
    wflat = winner(vid_idx)
    vid3 = vid_idx.reshape(_NW, _NCHUNK, _CHUNK)
    out_ref = jax.new_ref(queue.reshape(_N_VIDEO, _P, _PL))
    upd = gather_blend(iflat, vid3, wflat, out_ref)
    scatter(upd, vid3, out_ref)
    return out_ref[...].reshape(_N_VIDEO, _N_MU, _OUT_DIM)


# speedup vs baseline: 1.0162x; 1.0162x over previous
import functools

import jax
import jax.numpy as jnp
from jax import lax
from jax.experimental import pallas as pl
from jax.experimental.pallas import tpu as pltpu
from jax.experimental.pallas import tpu_sc as plsc

_N_VIDEO = 100000
_N_MU = 8
_OUT_DIM = 64
_BATCH = 16384
_ROW = _N_MU * _OUT_DIM
_MOM = 0.9

_NC = 2
_NS = 16
_NW = _NC * _NS
_B_PER_W = _BATCH // _NW
_CHUNK = 64
_NCHUNK = _B_PER_W // _CHUNK
_LANE = 16

_V_BLOCK = 4096
_NIDV = _BATCH // _LANE


def _worker_id():
    return lax.axis_index("s") * _NC + lax.axis_index("c")


def _winner_body(vid_hbm, w_hbm, vidx_v, wtab_v):
    w = _worker_id()
    lo = w * _V_BLOCK
    pltpu.sync_copy(vid_hbm, vidx_v)

    neg1 = jnp.full((_LANE,), -1, jnp.int32)

    @pl.loop(0, _V_BLOCK // _LANE)
    def _init(i):
        wtab_v[pl.ds(i * _LANE, _LANE)] = neg1

    iota = lax.iota(jnp.int32, _LANE)

    @pl.loop(0, _NIDV)
    def _scan(i):
        v = vidx_v[pl.ds(i * _LANE, _LANE)]
        b = i * _LANE + iota
        r = v - lo
        m = (r >= 0) & (r < _V_BLOCK)
        r = jnp.where(m, r, 0)
        _, last = plsc.scan_count(v, m)
        plsc.store_scatter(wtab_v, [r], b, mask=m & last)

    pltpu.sync_copy(wtab_v, w_hbm.at[pl.ds(lo, _V_BLOCK)])


def _gather_blend_body(q_hbm, i_hbm, vid_hbm, wflat_hbm, upd_hbm,
                       idx_v, bw_v, qbuf, ibuf, gsem, isem, wsem, ssem):
    w = _worker_id()
    base = w * _B_PER_W
    pltpu.sync_copy(vid_hbm.at[w], idx_v)

    for j in range(_NCHUNK):
        cq = pltpu.async_copy(q_hbm.at[idx_v.at[j]], qbuf, gsem)
        pltpu.async_copy(wflat_hbm.at[idx_v.at[j]], bw_v, wsem).wait()
        ci = pltpu.async_copy(i_hbm.at[bw_v], ibuf, isem)
        cq.wait()
        ci.wait()

        @pl.loop(0, _CHUNK * _ROW // _LANE)
        def _blend(i):
            r = i // (_ROW // _LANE)
            c = (i % (_ROW // _LANE)) * _LANE
            q = qbuf[r, pl.ds(c, _LANE)]
            x = ibuf[r, pl.ds(c, _LANE)]
            qbuf[r, pl.ds(c, _LANE)] = q * _MOM + x * (1.0 - _MOM)

        pltpu.async_copy(
            qbuf, upd_hbm.at[pl.ds(base + j * _CHUNK, _CHUNK)], ssem).wait()


def _scatter_body(upd_hbm, vid_hbm, q_ref, idx_v, buf, ssem):
    w = _worker_id()
    base = w * _B_PER_W
    pltpu.sync_copy(vid_hbm.at[w], idx_v)

    for j in range(_NCHUNK):
        pltpu.sync_copy(upd_hbm.at[pl.ds(base + j * _CHUNK, _CHUNK)], buf)
        pltpu.async_copy(buf, q_ref.at[idx_v.at[j]], ssem).wait()


@functools.cache
def _get_kernels():
    mesh = plsc.VectorSubcoreMesh(
        core_axis_name="c", subcore_axis_name="s", num_cores=_NC,
        num_subcores=_NS)
    winner = pl.kernel(
        _winner_body,
        out_type=jax.ShapeDtypeStruct((_NW * _V_BLOCK,), jnp.int32),
        mesh=mesh,
        compiler_params=pltpu.CompilerParams(needs_layout_passes=False),
        scratch_types=[
            pltpu.VMEM((_BATCH,), jnp.int32),
            pltpu.VMEM((_V_BLOCK,), jnp.int32),
        ],
    )
    gather_blend = pl.kernel(
        _gather_blend_body,
        out_type=jax.ShapeDtypeStruct((_BATCH, _ROW), jnp.float32),
        mesh=mesh,
        scratch_types=[
            pltpu.VMEM((_NCHUNK, _CHUNK), jnp.int32),
            pltpu.VMEM((_CHUNK,), jnp.int32),
            pltpu.VMEM((_CHUNK, _ROW), jnp.float32),
            pltpu.VMEM((_CHUNK, _ROW), jnp.float32),
            pltpu.SemaphoreType.DMA,
            pltpu.SemaphoreType.DMA,
            pltpu.SemaphoreType.DMA,
            pltpu.SemaphoreType.DMA,
        ],
    )
    scatter = pl.kernel(
        _scatter_body,
        out_type=(),
        mesh=mesh,
        scratch_types=[
            pltpu.VMEM((_NCHUNK, _CHUNK), jnp.int32),
            pltpu.VMEM((_CHUNK, _ROW), jnp.float32),
            pltpu.SemaphoreType.DMA,
        ],
    )
    return winner, gather_blend, scatter


@jax.jit
def kernel(queue, inp, vid_idx):
    winner, gather_blend, scatter = _get_kernels()
    qflat = queue.reshape(_N_VIDEO, _ROW)
    iflat = inp.reshape(_BATCH, _ROW)
    wflat = winner(vid_idx)
    vid3 = vid_idx.reshape(_NW, _NCHUNK, _CHUNK)
    upd = gather_blend(qflat, iflat, vid3, wflat)
    out_ref = jax.new_ref(qflat)
    scatter(upd, vid3, out_ref)
    return out_ref[...].reshape(_N_VIDEO, _N_MU, _OUT_DIM)


# speedup vs baseline: 1.0489x; 1.0322x over previous
import functools

import jax
import jax.numpy as jnp
from jax import lax
from jax.experimental import pallas as pl
from jax.experimental.pallas import tpu as pltpu
from jax.experimental.pallas import tpu_sc as plsc

_N_VIDEO = 100000
_N_MU = 8
_OUT_DIM = 64
_BATCH = 16384
_ROW = _N_MU * _OUT_DIM
_MOM = 0.9

_NC = 2
_NS = 16
_NW = _NC * _NS
_B_PER_W = _BATCH // _NW
_CHUNK = 32
_NCHUNK = _B_PER_W // _CHUNK
_LANE = 16

_V_BLOCK = 4096
_NIDV = _BATCH // _LANE


def _worker_id():
    return lax.axis_index("s") * _NC + lax.axis_index("c")


def _winner_body(vid_hbm, w_hbm, vidx_v, wtab_v):
    w = _worker_id()
    lo = w * _V_BLOCK
    pltpu.sync_copy(vid_hbm, vidx_v)

    neg1 = jnp.full((_LANE,), -1, jnp.int32)

    @pl.loop(0, _V_BLOCK // _LANE)
    def _init(i):
        wtab_v[pl.ds(i * _LANE, _LANE)] = neg1

    iota = lax.iota(jnp.int32, _LANE)

    @pl.loop(0, _NIDV)
    def _scan(i):
        v = vidx_v[pl.ds(i * _LANE, _LANE)]
        b = i * _LANE + iota
        r = v - lo
        m = (r >= 0) & (r < _V_BLOCK)
        r = jnp.where(m, r, 0)
        _, last = plsc.scan_count(v, m)
        plsc.store_scatter(wtab_v, [r], b, mask=m & last)

    pltpu.sync_copy(wtab_v, w_hbm.at[pl.ds(lo, _V_BLOCK)])


def _gather_blend_body(q_hbm, i_hbm, vid_hbm, wflat_hbm, upd_hbm,
                       idx_v, bw_v,
                       qb0, ib0, qb1, ib1,
                       wsem, gs0, is0, ss0, gs1, is1, ss1):
    w = _worker_id()
    base = w * _B_PER_W
    pltpu.sync_copy(vid_hbm.at[w], idx_v)
    for j in range(_NCHUNK):
        pltpu.async_copy(wflat_hbm.at[idx_v.at[j]], bw_v.at[j], wsem)
    for j in range(_NCHUNK):
        pltpu.make_async_copy(
            wflat_hbm.at[idx_v.at[j]], bw_v.at[j], wsem).wait()

    sets = ((qb0, ib0, gs0, is0, ss0), (qb1, ib1, gs1, is1, ss1))

    def issue(j):
        qb, ib, gs, isem, _ = sets[j % 2]
        pltpu.async_copy(q_hbm.at[idx_v.at[j]], qb, gs)
        pltpu.async_copy(i_hbm.at[bw_v.at[j]], ib, isem)

    issue(0)
    issue(1)
    for j in range(_NCHUNK):
        qb, ib, gs, isem, ss = sets[j % 2]
        pltpu.make_async_copy(q_hbm.at[idx_v.at[j]], qb, gs).wait()
        pltpu.make_async_copy(i_hbm.at[bw_v.at[j]], ib, isem).wait()

        @pl.loop(0, _CHUNK * _ROW // _LANE)
        def _blend(i):
            r = i // (_ROW // _LANE)
            c = (i % (_ROW // _LANE)) * _LANE
            q = qb[r, pl.ds(c, _LANE)]
            x = ib[r, pl.ds(c, _LANE)]
            qb[r, pl.ds(c, _LANE)] = q * _MOM + x * (1.0 - _MOM)

        st = pltpu.async_copy(
            qb, upd_hbm.at[pl.ds(base + j * _CHUNK, _CHUNK)], ss)
        if j + 2 < _NCHUNK:
            st.wait()
            issue(j + 2)
        else:
            st.wait()


def _scatter_body(upd_hbm, vid_hbm, q_ref, idx_v, buf, ssem):
    w = _worker_id()
    base = w * _B_PER_W
    pltpu.sync_copy(vid_hbm.at[w], idx_v)

    for j in range(_NCHUNK):
        pltpu.sync_copy(upd_hbm.at[pl.ds(base + j * _CHUNK, _CHUNK)], buf)
        pltpu.async_copy(buf, q_ref.at[idx_v.at[j]], ssem).wait()


@functools.cache
def _get_kernels():
    mesh = plsc.VectorSubcoreMesh(
        core_axis_name="c", subcore_axis_name="s", num_cores=_NC,
        num_subcores=_NS)
    winner = pl.kernel(
        _winner_body,
        out_type=jax.ShapeDtypeStruct((_NW * _V_BLOCK,), jnp.int32),
        mesh=mesh,
        compiler_params=pltpu.CompilerParams(needs_layout_passes=False),
        scratch_types=[
            pltpu.VMEM((_BATCH,), jnp.int32),
            pltpu.VMEM((_V_BLOCK,), jnp.int32),
        ],
    )
    gather_blend = pl.kernel(
        _gather_blend_body,
        out_type=jax.ShapeDtypeStruct((_BATCH, _ROW), jnp.float32),
        mesh=mesh,
        scratch_types=[
            pltpu.VMEM((_NCHUNK, _CHUNK), jnp.int32),
            pltpu.VMEM((_NCHUNK, _CHUNK), jnp.int32),
            pltpu.VMEM((_CHUNK, _ROW), jnp.float32),
            pltpu.VMEM((_CHUNK, _ROW), jnp.float32),
            pltpu.VMEM((_CHUNK, _ROW), jnp.float32),
            pltpu.VMEM((_CHUNK, _ROW), jnp.float32),
            pltpu.SemaphoreType.DMA,
            pltpu.SemaphoreType.DMA,
            pltpu.SemaphoreType.DMA,
            pltpu.SemaphoreType.DMA,
            pltpu.SemaphoreType.DMA,
            pltpu.SemaphoreType.DMA,
            pltpu.SemaphoreType.DMA,
        ],
    )
    scatter = pl.kernel(
        _scatter_body,
        out_type=(),
        mesh=mesh,
        scratch_types=[
            pltpu.VMEM((_NCHUNK, _CHUNK), jnp.int32),
            pltpu.VMEM((_CHUNK, _ROW), jnp.float32),
            pltpu.SemaphoreType.DMA,
        ],
    )
    return winner, gather_blend, scatter


@jax.jit
def kernel(queue, inp, vid_idx):
    winner, gather_blend, scatter = _get_kernels()
    qflat = queue.reshape(_N_VIDEO, _ROW)
    iflat = inp.reshape(_BATCH, _ROW)
    wflat = winner(vid_idx)
    vid3 = vid_idx.reshape(_NW, _NCHUNK, _CHUNK)
    upd = gather_blend(qflat, iflat, vid3, wflat)
    out_ref = jax.new_ref(qflat)
    scatter(upd, vid3, out_ref)
    return out_ref[...].reshape(_N_VIDEO, _N_MU, _OUT_DIM)


# speedup vs baseline: 1.0660x; 1.0163x over previous
import functools

import jax
import jax.numpy as jnp
from jax import lax
from jax.experimental import pallas as pl
from jax.experimental.pallas import tpu as pltpu
from jax.experimental.pallas import tpu_sc as plsc

_N_VIDEO = 100000
_N_MU = 8
_OUT_DIM = 64
_BATCH = 16384
_ROW = _N_MU * _OUT_DIM
_MOM = 0.9

_NC = 2
_NS = 16
_NW = _NC * _NS
_B_PER_W = _BATCH // _NW
_CHUNK = 32
_NCHUNK = _B_PER_W // _CHUNK
_LANE = 16

_V_BLOCK = 4096
_NIDV = _BATCH // _LANE


def _worker_id():
    return lax.axis_index("s") * _NC + lax.axis_index("c")


def _winner_body(vid_hbm, w_hbm, vidx_v, wtab_v):
    w = _worker_id()
    lo = w * _V_BLOCK
    pltpu.sync_copy(vid_hbm, vidx_v)

    neg1 = jnp.full((_LANE,), -1, jnp.int32)

    @pl.loop(0, _V_BLOCK // _LANE)
    def _init(i):
        wtab_v[pl.ds(i * _LANE, _LANE)] = neg1

    iota = lax.iota(jnp.int32, _LANE)

    @pl.loop(0, _NIDV, unroll=4)
    def _scan(i):
        v = vidx_v[pl.ds(i * _LANE, _LANE)]
        b = i * _LANE + iota
        r = v - lo
        m = (r >= 0) & (r < _V_BLOCK)
        r = jnp.where(m, r, 0)
        _, last = plsc.scan_count(v, m)
        plsc.store_scatter(wtab_v, [r], b, mask=m & last)

    pltpu.sync_copy(wtab_v, w_hbm.at[pl.ds(lo, _V_BLOCK)])


def _gather_blend_body(q_hbm, i_hbm, vid_hbm, wflat_hbm, upd_hbm,
                       idx_v, bw_v,
                       qb0, ib0, qb1, ib1,
                       wsem, gs0, is0, ss0, gs1, is1, ss1):
    w = _worker_id()
    base = w * _B_PER_W
    pltpu.sync_copy(vid_hbm.at[w], idx_v)
    for j in range(_NCHUNK):
        pltpu.async_copy(wflat_hbm.at[idx_v.at[j]], bw_v.at[j], wsem)
    for j in range(_NCHUNK):
        pltpu.make_async_copy(
            wflat_hbm.at[idx_v.at[j]], bw_v.at[j], wsem).wait()

    sets = ((qb0, ib0, gs0, is0, ss0), (qb1, ib1, gs1, is1, ss1))

    def issue(j):
        qb, ib, gs, isem, _ = sets[j % 2]
        pltpu.async_copy(q_hbm.at[idx_v.at[j]], qb, gs)
        pltpu.async_copy(i_hbm.at[bw_v.at[j]], ib, isem)

    issue(0)
    issue(1)
    for j in range(_NCHUNK):
        qb, ib, gs, isem, ss = sets[j % 2]
        pltpu.make_async_copy(q_hbm.at[idx_v.at[j]], qb, gs).wait()
        pltpu.make_async_copy(i_hbm.at[bw_v.at[j]], ib, isem).wait()

        @pl.loop(0, _CHUNK * _ROW // _LANE)
        def _blend(i):
            r = i // (_ROW // _LANE)
            c = (i % (_ROW // _LANE)) * _LANE
            q = qb[r, pl.ds(c, _LANE)]
            x = ib[r, pl.ds(c, _LANE)]
            qb[r, pl.ds(c, _LANE)] = q * _MOM + x * (1.0 - _MOM)

        st = pltpu.async_copy(
            qb, upd_hbm.at[pl.ds(base + j * _CHUNK, _CHUNK)], ss)
        if j + 2 < _NCHUNK:
            st.wait()
            issue(j + 2)
        else:
            st.wait()


def _scatter_body(upd_hbm, vid_hbm, q_ref, idx_v, buf0, buf1,
                  rs0, rs1, ws0, ws1):
    w = _worker_id()
    base = w * _B_PER_W
    pltpu.sync_copy(vid_hbm.at[w], idx_v)
    bufs = ((buf0, rs0, ws0), (buf1, rs1, ws1))

    def load(j):
        buf, rs, _ = bufs[j % 2]
        pltpu.async_copy(upd_hbm.at[pl.ds(base + j * _CHUNK, _CHUNK)], buf, rs)

    load(0)
    load(1)
    for j in range(_NCHUNK):
        buf, rs, ws = bufs[j % 2]
        pltpu.make_async_copy(
            upd_hbm.at[pl.ds(base + j * _CHUNK, _CHUNK)], buf, rs).wait()
        st = pltpu.async_copy(buf, q_ref.at[idx_v.at[j]], ws)
        st.wait()
        if j + 2 < _NCHUNK:
            load(j + 2)


@functools.cache
def _get_kernels():
    mesh = plsc.VectorSubcoreMesh(
        core_axis_name="c", subcore_axis_name="s", num_cores=_NC,
        num_subcores=_NS)
    winner = pl.kernel(
        _winner_body,
        out_type=jax.ShapeDtypeStruct((_NW * _V_BLOCK,), jnp.int32),
        mesh=mesh,
        compiler_params=pltpu.CompilerParams(needs_layout_passes=False),
        scratch_types=[
            pltpu.VMEM((_BATCH,), jnp.int32),
            pltpu.VMEM((_V_BLOCK,), jnp.int32),
        ],
    )
    gather_blend = pl.kernel(
        _gather_blend_body,
        out_type=jax.ShapeDtypeStruct((_BATCH, _ROW), jnp.float32),
        mesh=mesh,
        scratch_types=[
            pltpu.VMEM((_NCHUNK, _CHUNK), jnp.int32),
            pltpu.VMEM((_NCHUNK, _CHUNK), jnp.int32),
            pltpu.VMEM((_CHUNK, _ROW), jnp.float32),
            pltpu.VMEM((_CHUNK, _ROW), jnp.float32),
            pltpu.VMEM((_CHUNK, _ROW), jnp.float32),
            pltpu.VMEM((_CHUNK, _ROW), jnp.float32),
            pltpu.SemaphoreType.DMA,
            pltpu.SemaphoreType.DMA,
            pltpu.SemaphoreType.DMA,
            pltpu.SemaphoreType.DMA,
            pltpu.SemaphoreType.DMA,
            pltpu.SemaphoreType.DMA,
            pltpu.SemaphoreType.DMA,
        ],
    )
    scatter = pl.kernel(
        _scatter_body,
        out_type=(),
        mesh=mesh,
        scratch_types=[
            pltpu.VMEM((_NCHUNK, _CHUNK), jnp.int32),
            pltpu.VMEM((_CHUNK, _ROW), jnp.float32),
            pltpu.VMEM((_CHUNK, _ROW), jnp.float32),
            pltpu.SemaphoreType.DMA,
            pltpu.SemaphoreType.DMA,
            pltpu.SemaphoreType.DMA,
            pltpu.SemaphoreType.DMA,
        ],
    )
    return winner, gather_blend, scatter


@jax.jit
def kernel(queue, inp, vid_idx):
    winner, gather_blend, scatter = _get_kernels()
    qflat = queue.reshape(_N_VIDEO, _ROW)
    iflat = inp.reshape(_BATCH, _ROW)
    wflat = winner(vid_idx)
    vid3 = vid_idx.reshape(_NW, _NCHUNK, _CHUNK)
    upd = gather_blend(qflat, iflat, vid3, wflat)
    out_ref = jax.new_ref(qflat)
    scatter(upd, vid3, out_ref)
    return out_ref[...].reshape(_N_VIDEO, _N_MU, _OUT_DIM)


# speedup vs baseline: 1.0668x; 1.0007x over previous
import functools

import jax
import jax.numpy as jnp
from jax import lax
from jax.experimental import pallas as pl
from jax.experimental.pallas import tpu as pltpu
from jax.experimental.pallas import tpu_sc as plsc

_N_VIDEO = 100000
_N_MU = 8
_OUT_DIM = 64
_BATCH = 16384
_ROW = _N_MU * _OUT_DIM
_MOM = 0.9

_NC = 2
_NS = 16
_NW = _NC * _NS
_B_PER_W = _BATCH // _NW
_CHUNK = 32
_NCHUNK = _B_PER_W // _CHUNK
_LANE = 16

_V_BLOCK = 4096
_NIDV = _BATCH // _LANE


def _worker_id():
    return lax.axis_index("s") * _NC + lax.axis_index("c")


def _winner_body(vid_hbm, w_hbm, vidx_v, wtab_v):
    w = _worker_id()
    lo = w * _V_BLOCK
    pltpu.sync_copy(vid_hbm, vidx_v)

    neg1 = jnp.full((_LANE,), -1, jnp.int32)

    @pl.loop(0, _V_BLOCK // _LANE)
    def _init(i):
        wtab_v[pl.ds(i * _LANE, _LANE)] = neg1

    iota = lax.iota(jnp.int32, _LANE)

    @pl.loop(0, _NIDV, unroll=4)
    def _scan(i):
        v = vidx_v[pl.ds(i * _LANE, _LANE)]
        b = i * _LANE + iota
        r = v - lo
        m = (r >= 0) & (r < _V_BLOCK)
        r = jnp.where(m, r, 0)
        _, last = plsc.scan_count(v, m)
        plsc.store_scatter(wtab_v, [r], b, mask=m & last)

    pltpu.sync_copy(wtab_v, w_hbm.at[pl.ds(lo, _V_BLOCK)])


def _gather_blend_body(q_hbm, i_hbm, vid_hbm, wflat_hbm, upd_hbm,
                       idx_v, bw_v,
                       qb0, ib0, qb1, ib1,
                       wsem, gs0, is0, ss0, gs1, is1, ss1):
    w = _worker_id()
    base = w * _B_PER_W
    pltpu.sync_copy(vid_hbm.at[w], idx_v)
    for j in range(_NCHUNK):
        pltpu.async_copy(wflat_hbm.at[idx_v.at[j]], bw_v.at[j], wsem)
    for j in range(_NCHUNK):
        pltpu.make_async_copy(
            wflat_hbm.at[idx_v.at[j]], bw_v.at[j], wsem).wait()

    sets = ((qb0, ib0, gs0, is0, ss0), (qb1, ib1, gs1, is1, ss1))

    def issue(j):
        qb, ib, gs, isem, _ = sets[j % 2]
        pltpu.async_copy(q_hbm.at[idx_v.at[j]], qb, gs)
        pltpu.async_copy(i_hbm.at[bw_v.at[j]], ib, isem)

    issue(0)
    issue(1)
    for j in range(_NCHUNK):
        qb, ib, gs, isem, ss = sets[j % 2]
        pltpu.make_async_copy(q_hbm.at[idx_v.at[j]], qb, gs).wait()
        pltpu.make_async_copy(i_hbm.at[bw_v.at[j]], ib, isem).wait()

        @pl.loop(0, _CHUNK * _ROW // _LANE)
        def _blend(i):
            r = i // (_ROW // _LANE)
            c = (i % (_ROW // _LANE)) * _LANE
            q = qb[r, pl.ds(c, _LANE)]
            x = ib[r, pl.ds(c, _LANE)]
            qb[r, pl.ds(c, _LANE)] = q * _MOM + x * (1.0 - _MOM)

        st = pltpu.async_copy(
            qb, upd_hbm.at[pl.ds(base + j * _CHUNK, _CHUNK)], ss)
        if j + 2 < _NCHUNK:
            st.wait()
            issue(j + 2)
        else:
            st.wait()


def _scatter_body(upd_hbm, vid_hbm, q_ref, idx_v, buf0, buf1,
                  rs0, rs1, ws0, ws1):
    w = _worker_id()
    base = w * _B_PER_W
    pltpu.sync_copy(vid_hbm.at[w], idx_v)
    bufs = ((buf0, rs0, ws0), (buf1, rs1, ws1))

    def load(j):
        buf, rs, _ = bufs[j % 2]
        pltpu.async_copy(upd_hbm.at[pl.ds(base + j * _CHUNK, _CHUNK)], buf, rs)

    load(0)
    load(1)
    for j in range(_NCHUNK):
        buf, rs, ws = bufs[j % 2]
        pltpu.make_async_copy(
            upd_hbm.at[pl.ds(base + j * _CHUNK, _CHUNK)], buf, rs).wait()
        st = pltpu.async_copy(buf, q_ref.at[idx_v.at[j]], ws)
        st.wait()
        if j + 2 < _NCHUNK:
            load(j + 2)


@functools.cache
def _get_kernels():
    mesh = plsc.VectorSubcoreMesh(
        core_axis_name="c", subcore_axis_name="s", num_cores=_NC,
        num_subcores=_NS)
    winner = pl.kernel(
        _winner_body,
        out_type=jax.ShapeDtypeStruct((_NW * _V_BLOCK,), jnp.int32),
        mesh=mesh,
        compiler_params=pltpu.CompilerParams(needs_layout_passes=False),
        scratch_types=[
            pltpu.VMEM((_BATCH,), jnp.int32),
            pltpu.VMEM((_V_BLOCK,), jnp.int32),
        ],
    )
    gather_blend = pl.kernel(
        _gather_blend_body,
        out_type=jax.ShapeDtypeStruct((_BATCH, _ROW), jnp.float32),
        mesh=mesh,
        scratch_types=[
            pltpu.VMEM((_NCHUNK, _CHUNK), jnp.int32),
            pltpu.VMEM((_NCHUNK, _CHUNK), jnp.int32),
            pltpu.VMEM((_CHUNK, _ROW), jnp.float32),
            pltpu.VMEM((_CHUNK, _ROW), jnp.float32),
            pltpu.VMEM((_CHUNK, _ROW), jnp.float32),
            pltpu.VMEM((_CHUNK, _ROW), jnp.float32),
            pltpu.SemaphoreType.DMA,
            pltpu.SemaphoreType.DMA,
            pltpu.SemaphoreType.DMA,
            pltpu.SemaphoreType.DMA,
            pltpu.SemaphoreType.DMA,
            pltpu.SemaphoreType.DMA,
            pltpu.SemaphoreType.DMA,
        ],
    )
    scatter = pl.kernel(
        _scatter_body,
        out_type=(),
        mesh=mesh,
        scratch_types=[
            pltpu.VMEM((_NCHUNK, _CHUNK), jnp.int32),
            pltpu.VMEM((_CHUNK, _ROW), jnp.float32),
            pltpu.VMEM((_CHUNK, _ROW), jnp.float32),
            pltpu.SemaphoreType.DMA,
            pltpu.SemaphoreType.DMA,
            pltpu.SemaphoreType.DMA,
            pltpu.SemaphoreType.DMA,
        ],
    )
    return winner, gather_blend, scatter


@jax.jit
def kernel(queue, inp, vid_idx):
    winner, gather_blend, scatter = _get_kernels()
    qflat = queue.reshape(_N_VIDEO, _ROW)
    iflat = inp.reshape(_BATCH, _ROW)
    out_ref = jax.new_ref(qflat)
    wflat = winner(vid_idx)
    vid3 = vid_idx.reshape(_NW, _NCHUNK, _CHUNK)
    upd = gather_blend(qflat, iflat, vid3, wflat)
    scatter(upd, vid3, out_ref)
    return out_ref[...].reshape(_N_VIDEO, _N_MU, _OUT_DIM)
